# baseline (speedup 1.0000x reference)
import jax
import jax.numpy as jnp
from jax.experimental import pallas as pl
from jax.experimental.pallas import tpu as pltpu

_M, _N = 128, 32768
_RB = 64
_NB = _M // _RB
_NC = 2
_C = _N // _NC


def _quant_body(x_hbm, o_hbm, xb, ob, insem, outsem):
    def in_cp(rb, c):
        return pltpu.make_async_copy(
            x_hbm.at[pl.ds(rb * _RB, _RB), pl.ds(c * _C, _C)],
            xb.at[rb % 2, :, pl.ds(c * _C, _C)],
            insem.at[rb % 2, c],
        )

    def out_cp(rb, c):
        return pltpu.make_async_copy(
            ob.at[rb % 2, :, pl.ds(c * _C, _C)],
            o_hbm.at[pl.ds(rb * _RB, _RB), pl.ds(c * _C, _C)],
            outsem.at[rb % 2, c],
        )

    for rb in range(_NB):
        for c in range(_NC):
            in_cp(rb, c).start()

    for rb in range(_NB):
        nb = rb % 2
        acc = jnp.zeros((_RB, 1), jnp.float32)
        for c in range(_NC):
            in_cp(rb, c).wait()
            chunk = xb[nb, :, pl.ds(c * _C, _C)]
            acc = jnp.maximum(
                acc, jnp.max(jnp.abs(chunk), axis=-1, keepdims=True)
            )
        thr = 0.5 * (acc + 1e-6)
        nthr = -thr
        for c in range(_NC):
            chunk = xb[nb, :, pl.ds(c * _C, _C)]
            ob[nb, :, pl.ds(c * _C, _C)] = jnp.where(
                chunk > thr, 1.0, jnp.where(chunk < nthr, -1.0, 0.0)
            )
            out_cp(rb, c).start()

    for rb in range(_NB):
        for c in range(_NC):
            out_cp(rb, c).wait()


def kernel(x):
    return pl.pallas_call(
        _quant_body,
        in_specs=[pl.BlockSpec(memory_space=pl.ANY)],
        out_specs=pl.BlockSpec(memory_space=pl.ANY),
        out_shape=jax.ShapeDtypeStruct((_M, _N), x.dtype),
        scratch_shapes=[
            pltpu.VMEM((2, _RB, _N), jnp.float32),
            pltpu.VMEM((2, _RB, _N), jnp.float32),
            pltpu.SemaphoreType.DMA((2, _NC)),
            pltpu.SemaphoreType.DMA((2, _NC)),
        ],
    )(x)


# speedup vs baseline: 1.0210x; 1.0210x over previous
import jax
import jax.numpy as jnp
from jax.experimental import pallas as pl
from jax.experimental.pallas import tpu as pltpu

_M, _N = 128, 32768
_RB = 64
_NB = _M // _RB
_NC = 4
_C = _N // _NC


def _quant_body(x_hbm, o_hbm, xb, ob, insem, outsem):
    def in_cp(rb, c):
        return pltpu.make_async_copy(
            x_hbm.at[pl.ds(rb * _RB, _RB), pl.ds(c * _C, _C)],
            xb.at[rb % 2, :, pl.ds(c * _C, _C)],
            insem.at[rb % 2, c],
        )

    def out_cp(rb, c):
        return pltpu.make_async_copy(
            ob.at[rb % 2, :, pl.ds(c * _C, _C)],
            o_hbm.at[pl.ds(rb * _RB, _RB), pl.ds(c * _C, _C)],
            outsem.at[rb % 2, c],
        )

    for rb in range(_NB):
        for c in range(_NC):
            in_cp(rb, c).start()

    for rb in range(_NB):
        nb = rb % 2
        acc = jnp.zeros((_RB, 1), jnp.float32)
        for c in range(_NC):
            in_cp(rb, c).wait()
            chunk = xb[nb, :, pl.ds(c * _C, _C)]
            acc = jnp.maximum(
                acc, jnp.max(jnp.abs(chunk), axis=-1, keepdims=True)
            )
        thr = 0.5 * (acc + 1e-6)
        nthr = -thr
        for c in range(_NC):
            chunk = xb[nb, :, pl.ds(c * _C, _C)]
            ob[nb, :, pl.ds(c * _C, _C)] = jnp.where(
                chunk > thr, 1.0, jnp.where(chunk < nthr, -1.0, 0.0)
            )
            out_cp(rb, c).start()

    for rb in range(_NB):
        for c in range(_NC):
            out_cp(rb, c).wait()


def kernel(x):
    return pl.pallas_call(
        _quant_body,
        in_specs=[pl.BlockSpec(memory_space=pl.ANY)],
        out_specs=pl.BlockSpec(memory_space=pl.ANY),
        out_shape=jax.ShapeDtypeStruct((_M, _N), x.dtype),
        scratch_shapes=[
            pltpu.VMEM((2, _RB, _N), jnp.float32),
            pltpu.VMEM((2, _RB, _N), jnp.float32),
            pltpu.SemaphoreType.DMA((2, _NC)),
            pltpu.SemaphoreType.DMA((2, _NC)),
        ],
    )(x)
